# baseline (speedup 1.0000x reference)
import functools

import jax
import jax.numpy as jnp
from jax import lax
from jax.experimental import pallas as pl
from jax.experimental.pallas import tpu as pltpu
from jax.experimental.pallas import tpu_sc as plsc

_B = 10000
_K = 32
_D = 128

_NC = 2
_NS = 16
_CH = 128

_CH0 = 136
_CH1 = 32
_CT = _NS * (_CH0 + _CH1)
_N_PAD = _CT * _CH

_BB = 200
_GRID = _B // _BB
_UBLK = (_B * _K) // _BB


def _sc_gather_body(table_h, idx_h, out_h, idx_v, bufs, gsems, osems):
    c = lax.axis_index("c")
    s = lax.axis_index("s")

    def start_g(j, b):
        pltpu.make_async_copy(
            table_h.at[idx_v.at[j]], bufs.at[b], gsems.at[b]).start()

    def wait_g(b):
        pltpu.make_async_copy(
            table_h.at[idx_v.at[0]], bufs.at[b], gsems.at[b]).wait()

    def start_s(row0, b):
        pltpu.make_async_copy(
            bufs.at[b], out_h.at[pl.ds(row0, _CH)], osems.at[b]).start()

    def wait_s(b):
        pltpu.make_async_copy(
            bufs.at[b], out_h.at[pl.ds(0, _CH)], osems.at[b]).wait()

    def run(nch, base_chunk):
        pltpu.sync_copy(idx_h.at[pl.ds(base_chunk, nch)],
                        idx_v.at[pl.ds(0, nch)])
        base_row = base_chunk * _CH
        nsuper = nch // 4

        start_g(0, 0)
        start_g(1, 1)

        @pl.loop(0, nsuper)
        def _super(u):
            p0 = 4 * u
            p1 = 4 * u + 2
            wait_g(0)
            wait_g(1)

            @pl.when(u > 0)
            def _():
                wait_s(2)
                wait_s(3)

            start_g(p1, 2)
            start_g(p1 + 1, 3)
            start_s(base_row + p0 * _CH, 0)
            start_s(base_row + (p0 + 1) * _CH, 1)

            wait_g(2)
            wait_g(3)
            wait_s(0)
            wait_s(1)

            @pl.when(u < nsuper - 1)
            def _():
                start_g(p0 + 4, 0)
                start_g(p0 + 5, 1)

            start_s(base_row + p1 * _CH, 2)
            start_s(base_row + (p1 + 1) * _CH, 3)

        wait_s(2)
        wait_s(3)

    @pl.when(c == 0)
    def _core0():
        run(_CH0, s * _CH0)

    @pl.when(c == 1)
    def _core1():
        run(_CH1, _NS * _CH0 + s * _CH1)


@jax.jit
def _sc_gather(table, idx):
    mesh = plsc.VectorSubcoreMesh(core_axis_name="c", subcore_axis_name="s")
    k = pl.kernel(
        _sc_gather_body,
        out_type=jax.ShapeDtypeStruct((_N_PAD, _D), jnp.float32),
        mesh=mesh,
        scratch_types=[
            pltpu.VMEM((_CH0, _CH), jnp.int32),
            pltpu.VMEM((4, _CH, _D), jnp.float32),
            pltpu.SemaphoreType.DMA((4,)),
            pltpu.SemaphoreType.DMA((4,)),
        ],
    )
    return k(table, idx)


def _tc_mlp_body(e3_ref, u_ref, w1t_ref, w1b_ref, w2_ref, w3t_ref,
                 b1_ref, b2_ref, b3_ref, out_ref):
    e3 = e3_ref[...]
    e2 = e3.reshape(_BB * _K, _D)
    u = u_ref[...]

    uw = jnp.dot(u, w1b_ref[...], preferred_element_type=jnp.float32)
    uw = uw + b1_ref[...]
    z1 = jnp.dot(e2, w1t_ref[...], preferred_element_type=jnp.float32)
    h1 = jnp.maximum(z1.reshape(_BB, _K, _D) + uw[:, None, :], 0.0)

    h2 = jnp.dot(h1.reshape(_BB * _K, _D), w2_ref[...],
                 preferred_element_type=jnp.float32)
    h2 = jnp.maximum(h2 + b2_ref[...], 0.0)

    w3row = w3t_ref[...].reshape(1, 1, _D)
    t = jnp.sum(h2.reshape(_BB, _K, _D) * w3row, axis=2, keepdims=True)
    t = t + b3_ref[0, 0]

    m = jnp.max(t, axis=1, keepdims=True)
    p = jnp.exp(t - m)
    s = jnp.sum(p, axis=1, keepdims=True)
    att = p / s

    out_ref[...] = jnp.sum(e3 * att, axis=1)


def _tc_mlp(e3, u, w1t, w1b, w2, w3t, b1, b2, b3):
    return pl.pallas_call(
        _tc_mlp_body,
        grid=(_GRID,),
        in_specs=[
            pl.BlockSpec((_BB, _K, _D), lambda i: (i, 0, 0)),
            pl.BlockSpec((_BB, _D), lambda i: (i + _UBLK, 0)),
            pl.BlockSpec((_D, _D), lambda i: (0, 0)),
            pl.BlockSpec((_D, _D), lambda i: (0, 0)),
            pl.BlockSpec((_D, _D), lambda i: (0, 0)),
            pl.BlockSpec((1, _D), lambda i: (0, 0)),
            pl.BlockSpec((1, _D), lambda i: (0, 0)),
            pl.BlockSpec((1, _D), lambda i: (0, 0)),
            pl.BlockSpec((1, 1), lambda i: (0, 0)),
        ],
        out_specs=pl.BlockSpec((_BB, _D), lambda i: (i, 0)),
        out_shape=jax.ShapeDtypeStruct((_B, _D), jnp.float32),
    )(e3, u, w1t, w1b, w2, w3t, b1, b2, b3)


def kernel(nodes, to_neighs, u2e, W1, b1, W2, b2, W3, b3):
    idx = jnp.zeros((_N_PAD,), jnp.int32)
    idx = idx.at[: _B * _K].set(to_neighs.reshape(-1))
    idx = idx.at[_B * _K: _B * _K + _B].set(nodes)
    idx = idx.reshape(_CT, _CH)

    rows = _sc_gather(u2e, idx)
    e3 = rows.reshape(_N_PAD // _K, _K, _D)

    return _tc_mlp(e3, rows, W1[:_D], W1[_D:], W2, W3.reshape(1, _D),
                   b1.reshape(1, _D), b2.reshape(1, _D), b3.reshape(1, 1))
